# Initial kernel scaffold; baseline (speedup 1.0000x reference)
import jax
import jax.numpy as jnp
from jax.experimental import pallas as pl


def kernel(x_author, x_paper, edge_index_writes, edge_index_written_by, params):
    raise NotImplementedError("write your pallas kernel here")


# speedup vs baseline: 2.1214x; 2.1214x over previous
import functools

import jax
import jax.numpy as jnp
from jax import lax
from jax.experimental import pallas as pl
from jax.experimental.pallas import tpu as pltpu
from jax.experimental.pallas import tpu_sc as plsc

N_NODE = 50000
E = 600000
D_IN = 128
H = 256
N_LAYER = 3

NC = 2
NT = 16
LANES = 16

W_SLAB = 32
IDXW = 128
RB = 4
ROWS_PER_TILE = 296
N_BATCH = ROWS_PER_TILE // RB
EROWS = ROWS_PER_TILE * NT
E_PAD = EROWS * IDXW
ACC_ROWS = N_NODE + LANES
WB_ROWS = N_NODE // NT

_mesh = plsc.VectorSubcoreMesh(core_axis_name="c", subcore_axis_name="s")
_sc_params = pltpu.CompilerParams(use_tc_tiling_on_sc=False)


def _segsum_body(nsd, table_hbm, src_hbm, dst_hbm, zeros_hbm, out_hbm,
                 srcraw_v, gidx_v, dst_v, rows_v, acc, sem):
    c = lax.axis_index("c")
    t = lax.axis_index("s")
    nsh = nsd // NC
    wb0 = t * WB_ROWS

    for p in range(nsh):
        s = c * nsh + p
        pltpu.sync_copy(zeros_hbm, acc.at[pl.ds(wb0, WB_ROWS)])
        plsc.subcore_barrier()

        def batch(b, carry):
            r0 = t * ROWS_PER_TILE + b * RB
            pltpu.sync_copy(dst_hbm.at[pl.ds(r0, RB)], dst_v)
            pltpu.sync_copy(src_hbm.at[pl.ds(r0 * IDXW, RB * IDXW)], srcraw_v)

            def adjust(k, carry2):
                v = srcraw_v[pl.ds(k * LANES, LANES)]
                gidx_v[pl.ds(k * LANES, LANES)] = v * nsd + s
                return carry2

            lax.fori_loop(0, RB * IDXW // LANES, adjust, 0)
            for j in range(RB):
                pltpu.async_copy(
                    table_hbm.at[gidx_v.at[pl.ds(j * IDXW, IDXW)]],
                    rows_v, sem).wait()
                pltpu.sync_copy(rows_v, acc.at[dst_v.at[j]], add=True)
            return carry

        lax.fori_loop(0, N_BATCH, batch, 0)
        plsc.subcore_barrier()
        pltpu.sync_copy(acc.at[pl.ds(wb0, WB_ROWS)],
                        out_hbm.at[pl.ds(wb0, WB_ROWS), s])
        plsc.subcore_barrier()


def _make_segsum(nsd):
    return pl.kernel(
        functools.partial(_segsum_body, nsd),
        out_type=jax.ShapeDtypeStruct((N_NODE, nsd, W_SLAB), jnp.float32),
        mesh=_mesh,
        scratch_types=[
            pltpu.VMEM((RB * IDXW,), jnp.int32),
            pltpu.VMEM((RB * IDXW,), jnp.int32),
            pltpu.VMEM((RB, IDXW), jnp.int32),
            pltpu.VMEM((IDXW, W_SLAB), jnp.float32),
            pltpu.VMEM_SHARED((ACC_ROWS, W_SLAB), jnp.float32),
            pltpu.SemaphoreType.DMA,
        ],
        compiler_params=_sc_params,
    )


def _count_body(dst_hbm, ones_hbm, zeros_hbm, out_hbm,
                dst_v, ones_v, acc, sem):
    c = lax.axis_index("c")
    t = lax.axis_index("s")
    wb0 = t * WB_ROWS
    rows_per_core = EROWS // NC
    rows_per_tile = rows_per_core // NT

    pltpu.sync_copy(zeros_hbm, acc.at[pl.ds(wb0, WB_ROWS)])
    pltpu.sync_copy(ones_hbm, ones_v)
    plsc.subcore_barrier()

    def batch(b, carry):
        r0 = c * rows_per_core + t * rows_per_tile + b
        pltpu.sync_copy(dst_hbm.at[pl.ds(r0, 1)], dst_v)
        pltpu.sync_copy(ones_v, acc.at[dst_v.at[0]], add=True)
        return carry

    lax.fori_loop(0, rows_per_tile, batch, 0)
    plsc.subcore_barrier()
    pltpu.sync_copy(acc.at[pl.ds(wb0, WB_ROWS)],
                    out_hbm.at[c, pl.ds(wb0, WB_ROWS)])


_count_kernel = pl.kernel(
    _count_body,
    out_type=jax.ShapeDtypeStruct((NC, N_NODE, W_SLAB), jnp.float32),
    mesh=_mesh,
    scratch_types=[
        pltpu.VMEM((1, IDXW), jnp.int32),
        pltpu.VMEM((IDXW, W_SLAB), jnp.float32),
        pltpu.VMEM_SHARED((ACC_ROWS, W_SLAB), jnp.float32),
        pltpu.SemaphoreType.DMA,
    ],
    compiler_params=_sc_params,
)


ROW_BLK = 2000


def _dense_body(sums_ref, cnt_ref, xd_ref, wl_ref, wr_ref, b_ref, out_ref):
    cnt = cnt_ref[0, :, 0:1] + cnt_ref[1, :, 0:1]
    inv = 1.0 / jnp.maximum(cnt, 1.0)
    agg = sums_ref[...] * inv
    out_ref[...] = (agg @ wl_ref[...] + xd_ref[...] @ wr_ref[...]
                    + b_ref[0, :])


def _dense_call(sums, cnt, xd, wl, wr, b):
    din = xd.shape[1]
    return pl.pallas_call(
        _dense_body,
        grid=(N_NODE // ROW_BLK,),
        in_specs=[
            pl.BlockSpec((ROW_BLK, din), lambda i: (i, 0)),
            pl.BlockSpec((NC, ROW_BLK, W_SLAB), lambda i: (0, i, 0)),
            pl.BlockSpec((ROW_BLK, din), lambda i: (i, 0)),
            pl.BlockSpec((din, H), lambda i: (0, 0)),
            pl.BlockSpec((din, H), lambda i: (0, 0)),
            pl.BlockSpec((1, H), lambda i: (0, 0)),
        ],
        out_specs=pl.BlockSpec((ROW_BLK, H), lambda i: (i, 0)),
        out_shape=jax.ShapeDtypeStruct((N_NODE, H), jnp.float32),
    )(sums, cnt, xd, wl, wr, b.reshape(1, H))


def _pad_edges(edge_index):
    src = edge_index[0].astype(jnp.int32)
    dst = edge_index[1].astype(jnp.int32)
    pad = E_PAD - E
    src = jnp.concatenate([src, jnp.zeros((pad,), jnp.int32)])
    dst = jnp.concatenate([dst, jnp.full((pad,), N_NODE, jnp.int32)])
    return src, dst.reshape(EROWS, IDXW)


def kernel(x_author, x_paper, edge_index_writes, edge_index_written_by, params):
    src_w, dst_w = _pad_edges(edge_index_writes)
    src_b, dst_b = _pad_edges(edge_index_written_by)
    zeros = jnp.zeros((WB_ROWS, W_SLAB), jnp.float32)
    ones = jnp.ones((IDXW, W_SLAB), jnp.float32)

    cnt_w = _count_kernel(dst_w, ones, zeros)
    cnt_b = _count_kernel(dst_b, ones, zeros)

    xa, xp = x_author, x_paper
    for i in range(N_LAYER):
        din = D_IN if i == 0 else H
        nsd = din // W_SLAB
        seg = _make_segsum(nsd)
        sums_p = seg(xa.reshape(-1, W_SLAB), src_w, dst_w, zeros)
        sums_a = seg(xp.reshape(-1, W_SLAB), src_b, dst_b, zeros)
        new_xp = _dense_call(sums_p.reshape(N_NODE, din), cnt_w, xp,
                             params[f"Wl_{i}_writes"], params[f"Wr_{i}_writes"],
                             params[f"b_{i}_writes"])
        new_xa = _dense_call(sums_a.reshape(N_NODE, din), cnt_b, xa,
                             params[f"Wl_{i}_written"], params[f"Wr_{i}_written"],
                             params[f"b_{i}_written"])
        xa, xp = new_xa, new_xp
    return (xa, xp)


# speedup vs baseline: 3.0425x; 1.4342x over previous
import functools

import jax
import jax.numpy as jnp
from jax import lax
from jax.experimental import pallas as pl
from jax.experimental.pallas import tpu as pltpu
from jax.experimental.pallas import tpu_sc as plsc

N_NODE = 50000
E = 600000
D_IN = 128
H = 256
N_LAYER = 3

NC = 2
NT = 16
LANES = 16

W_SLAB = 32
IDXW = 128
RB = 4
ROWS_PER_TILE = 296
N_BATCH = ROWS_PER_TILE // RB
EROWS = ROWS_PER_TILE * NT
E_PAD = EROWS * IDXW
ACC_ROWS = N_NODE + LANES
WB_ROWS = N_NODE // NT

_mesh = plsc.VectorSubcoreMesh(core_axis_name="c", subcore_axis_name="s")
_sc_params = pltpu.CompilerParams(use_tc_tiling_on_sc=False)


def _segsum_body(nsd, table_hbm, gidx_hbm, dst_hbm, zeros_hbm, out_hbm,
                 src_v, dst_v, rows_v, acc, gsem, ssem):
    c = lax.axis_index("c")
    t = lax.axis_index("s")
    nsh = nsd // NC
    wb0 = t * WB_ROWS
    tab_len = N_NODE * nsd - (nsd - 1)

    for p in range(nsh):
        s = c * nsh + p
        tab = table_hbm.at[pl.ds(s, tab_len)]
        pltpu.sync_copy(zeros_hbm, acc.at[pl.ds(wb0, WB_ROWS)])
        plsc.subcore_barrier()

        def batch(b, carry):
            r0 = t * ROWS_PER_TILE + b * RB
            pltpu.sync_copy(dst_hbm.at[pl.ds(r0, RB)], dst_v)
            pltpu.sync_copy(gidx_hbm.at[pl.ds(r0 * IDXW, RB * IDXW)], src_v)
            gathers = [
                pltpu.async_copy(tab.at[src_v.at[pl.ds(j * IDXW, IDXW)]],
                                 rows_v.at[j], gsem)
                for j in range(RB)
            ]
            scatters = []
            for j in range(RB):
                gathers[j].wait()
                scatters.append(
                    pltpu.async_copy(rows_v.at[j], acc.at[dst_v.at[j]],
                                     ssem, add=True))
            for d in scatters:
                d.wait()
            return carry

        lax.fori_loop(0, N_BATCH, batch, 0)
        plsc.subcore_barrier()
        pltpu.sync_copy(acc.at[pl.ds(wb0, WB_ROWS)],
                        out_hbm.at[pl.ds(wb0, WB_ROWS), s])
        plsc.subcore_barrier()


def _make_segsum(nsd):
    return pl.kernel(
        functools.partial(_segsum_body, nsd),
        out_type=jax.ShapeDtypeStruct((N_NODE, nsd, W_SLAB), jnp.float32),
        mesh=_mesh,
        scratch_types=[
            pltpu.VMEM((RB * IDXW,), jnp.int32),
            pltpu.VMEM((RB, IDXW), jnp.int32),
            pltpu.VMEM((RB, IDXW, W_SLAB), jnp.float32),
            pltpu.VMEM_SHARED((ACC_ROWS, W_SLAB), jnp.float32),
            pltpu.SemaphoreType.DMA,
            pltpu.SemaphoreType.DMA,
        ],
        compiler_params=_sc_params,
    )


def _count_body(dst_hbm, ones_hbm, zeros_hbm, out_hbm,
                dst_v, ones_v, acc, sem):
    c = lax.axis_index("c")
    t = lax.axis_index("s")
    wb0 = t * WB_ROWS
    rows_per_core = EROWS // NC
    rows_per_tile = rows_per_core // NT

    pltpu.sync_copy(zeros_hbm, acc.at[pl.ds(wb0, WB_ROWS)])
    pltpu.sync_copy(ones_hbm, ones_v)
    plsc.subcore_barrier()

    def batch(b, carry):
        r0 = c * rows_per_core + t * rows_per_tile + b
        pltpu.sync_copy(dst_hbm.at[pl.ds(r0, 1)], dst_v)
        pltpu.sync_copy(ones_v, acc.at[dst_v.at[0]], add=True)
        return carry

    lax.fori_loop(0, rows_per_tile, batch, 0)
    plsc.subcore_barrier()
    pltpu.sync_copy(acc.at[pl.ds(wb0, WB_ROWS)],
                    out_hbm.at[c, pl.ds(wb0, WB_ROWS)])


_count_kernel = pl.kernel(
    _count_body,
    out_type=jax.ShapeDtypeStruct((NC, N_NODE, W_SLAB), jnp.float32),
    mesh=_mesh,
    scratch_types=[
        pltpu.VMEM((1, IDXW), jnp.int32),
        pltpu.VMEM((IDXW, W_SLAB), jnp.float32),
        pltpu.VMEM_SHARED((ACC_ROWS, W_SLAB), jnp.float32),
        pltpu.SemaphoreType.DMA,
    ],
    compiler_params=_sc_params,
)


ROW_BLK = 2000


def _dense_body(sums_ref, cnt_ref, xd_ref, wl_ref, wr_ref, b_ref, out_ref):
    cnt = cnt_ref[0, :, 0:1] + cnt_ref[1, :, 0:1]
    inv = 1.0 / jnp.maximum(cnt, 1.0)
    agg = sums_ref[...] * inv
    out_ref[...] = (agg @ wl_ref[...] + xd_ref[...] @ wr_ref[...]
                    + b_ref[0, :])


def _dense_call(sums, cnt, xd, wl, wr, b):
    din = xd.shape[1]
    return pl.pallas_call(
        _dense_body,
        grid=(N_NODE // ROW_BLK,),
        in_specs=[
            pl.BlockSpec((ROW_BLK, din), lambda i: (i, 0)),
            pl.BlockSpec((NC, ROW_BLK, W_SLAB), lambda i: (0, i, 0)),
            pl.BlockSpec((ROW_BLK, din), lambda i: (i, 0)),
            pl.BlockSpec((din, H), lambda i: (0, 0)),
            pl.BlockSpec((din, H), lambda i: (0, 0)),
            pl.BlockSpec((1, H), lambda i: (0, 0)),
        ],
        out_specs=pl.BlockSpec((ROW_BLK, H), lambda i: (i, 0)),
        out_shape=jax.ShapeDtypeStruct((N_NODE, H), jnp.float32),
    )(sums, cnt, xd, wl, wr, b.reshape(1, H))


def _pad_edges(edge_index):
    src = edge_index[0].astype(jnp.int32)
    dst = edge_index[1].astype(jnp.int32)
    pad = E_PAD - E
    src = jnp.concatenate([src, jnp.zeros((pad,), jnp.int32)])
    dst = jnp.concatenate([dst, jnp.full((pad,), N_NODE, jnp.int32)])
    return src, dst.reshape(EROWS, IDXW)


def kernel(x_author, x_paper, edge_index_writes, edge_index_written_by, params):
    src_w, dst_w = _pad_edges(edge_index_writes)
    src_b, dst_b = _pad_edges(edge_index_written_by)
    gidx_w = {nsd: src_w * nsd for nsd in (D_IN // W_SLAB, H // W_SLAB)}
    gidx_b = {nsd: src_b * nsd for nsd in (D_IN // W_SLAB, H // W_SLAB)}
    zeros = jnp.zeros((WB_ROWS, W_SLAB), jnp.float32)
    ones = jnp.ones((IDXW, W_SLAB), jnp.float32)

    cnt_w = _count_kernel(dst_w, ones, zeros)
    cnt_b = _count_kernel(dst_b, ones, zeros)

    xa, xp = x_author, x_paper
    for i in range(N_LAYER):
        din = D_IN if i == 0 else H
        nsd = din // W_SLAB
        seg = _make_segsum(nsd)
        sums_p = seg(xa.reshape(-1, W_SLAB), gidx_w[nsd], dst_w, zeros)
        sums_a = seg(xp.reshape(-1, W_SLAB), gidx_b[nsd], dst_b, zeros)
        new_xp = _dense_call(sums_p.reshape(N_NODE, din), cnt_w, xp,
                             params[f"Wl_{i}_writes"], params[f"Wr_{i}_writes"],
                             params[f"b_{i}_writes"])
        new_xa = _dense_call(sums_a.reshape(N_NODE, din), cnt_b, xa,
                             params[f"Wl_{i}_written"], params[f"Wr_{i}_written"],
                             params[f"b_{i}_written"])
        xa, xp = new_xa, new_xp
    return (xa, xp)
